# baseline (speedup 1.0000x reference)
import jax
import jax.numpy as jnp
from jax.experimental import pallas as pl
from jax.experimental.pallas import tpu as pltpu


def _copy_body(k_in, v_in, k_out, v_out, k_sem, v_sem):
    k_copy = pltpu.make_async_copy(k_in, k_out, k_sem)
    v_copy = pltpu.make_async_copy(v_in, v_out, v_sem)
    k_copy.start()
    v_copy.start()
    k_copy.wait()
    v_copy.wait()


def kernel(new_k, new_v):
    out = pl.pallas_call(
        _copy_body,
        in_specs=[
            pl.BlockSpec(memory_space=pl.ANY),
            pl.BlockSpec(memory_space=pl.ANY),
        ],
        out_specs=[
            pl.BlockSpec(memory_space=pl.ANY),
            pl.BlockSpec(memory_space=pl.ANY),
        ],
        out_shape=[
            jax.ShapeDtypeStruct(new_k.shape, new_k.dtype),
            jax.ShapeDtypeStruct(new_v.shape, new_v.dtype),
        ],
        scratch_shapes=[pltpu.SemaphoreType.DMA, pltpu.SemaphoreType.DMA],
    )(new_k, new_v)
    return (out[0], out[1])
